# Initial kernel scaffold; baseline (speedup 1.0000x reference)
import jax
import jax.numpy as jnp
from jax.experimental import pallas as pl


def kernel(sparse_batch, dense_batch, stm_players, ft_w, ft_b, W1, b1, Wv, bv, Wp, bp):
    raise NotImplementedError("write your pallas kernel here")


# speedup vs baseline: 2.1252x; 2.1252x over previous
import functools

import jax
import jax.numpy as jnp
from jax import lax
from jax.experimental import pallas as pl
from jax.experimental.pallas import tpu as pltpu
from jax.experimental.pallas import tpu_sc as plsc

FT_DIM = 256
PIECE_HEX_DIM = 720
P1_CUTOFF = 360
NSPARSE = 6
CHUNK = 16


def _sc_ft_kernel(B, n_workers):
    per_w = B // n_workers
    n_chunks = per_w // CHUNK
    n_rows = CHUNK * NSPARSE
    mesh = plsc.VectorSubcoreMesh(core_axis_name="c", subcore_axis_name="s")

    @functools.partial(
        pl.kernel,
        mesh=mesh,
        out_type=jax.ShapeDtypeStruct((B, 2 * FT_DIM), jnp.float32),
        scratch_types=[
            pltpu.VMEM((NSPARSE, per_w), jnp.int32),
            pltpu.VMEM((per_w,), jnp.int32),
            pltpu.VMEM((FT_DIM,), jnp.float32),
            pltpu.VMEM((n_rows,), jnp.int32),
            pltpu.VMEM((n_rows, FT_DIM), jnp.float32),
            pltpu.VMEM((CHUNK, 2 * FT_DIM), jnp.float32),
            pltpu.SemaphoreType.DMA,
        ],
    )
    def k(sparse_hbm, stm_hbm, ftw_hbm, ftb_hbm, out_hbm,
          sp_v, stm_v, ftb_v, idx_v, rows_v, out_v, sem):
        nc = 2
        wid = lax.axis_index("s") * nc + lax.axis_index("c")
        pltpu.sync_copy(sparse_hbm.at[wid], sp_v)
        pltpu.sync_copy(stm_hbm.at[wid], stm_v)
        pltpu.sync_copy(ftb_hbm, ftb_v)

        def chunk_body(c, _):
            base = c * CHUNK
            stm1 = stm_v[pl.ds(base, CHUNK)]
            one = jnp.ones((CHUNK,), jnp.int32)
            zero_i = jnp.zeros((CHUNK,), jnp.int32)
            cnt_s = zero_i
            cnt_n = zero_i
            slots = [zero_i] * 6
            for j in range(NSPARSE):
                s = sp_v[j, pl.ds(base, CHUNK)]
                isp1 = jnp.where(s < P1_CUTOFF, one, zero_i)
                ist = isp1 ^ stm1
                sel_s = (ist == 1) & (cnt_s < 3)
                sel_n = (ist == 0) & (cnt_n < 3)
                for kk in range(3):
                    slots[kk] = jnp.where(sel_s & (cnt_s == kk), s, slots[kk])
                    slots[3 + kk] = jnp.where(sel_n & (cnt_n == kk), s,
                                              slots[3 + kk])
                cnt_s = cnt_s + ist
                cnt_n = cnt_n + (one - ist)
            for kk in range(6):
                idx_v[pl.ds(kk * CHUNK, CHUNK)] = slots[kk]
            pltpu.async_copy(ftw_hbm.at[idx_v], rows_v, sem).wait()

            def sample_body(b, _):
                for l in range(FT_DIM // 16):
                    sl = pl.ds(l * 16, 16)
                    bias = ftb_v[sl]
                    acc_s = (rows_v[b, sl] + rows_v[CHUNK + b, sl]
                             + rows_v[2 * CHUNK + b, sl] + bias)
                    acc_n = (rows_v[3 * CHUNK + b, sl]
                             + rows_v[4 * CHUNK + b, sl]
                             + rows_v[5 * CHUNK + b, sl] + bias)
                    zf = jnp.zeros((16,), jnp.float32)
                    out_v[b, sl] = jnp.maximum(acc_s, zf)
                    out_v[b, pl.ds(FT_DIM + l * 16, 16)] = jnp.maximum(acc_n, zf)
                return ()

            lax.fori_loop(0, CHUNK, sample_body, (), unroll=False)
            pltpu.sync_copy(out_v, out_hbm.at[pl.ds(wid * per_w + base, CHUNK), :])
            return ()

        lax.fori_loop(0, n_chunks, chunk_body, (), unroll=False)

    return k


def _tc_head_kernel(x1_ref, xd_ref, w1a_ref, w1d_ref, b1_ref, wv_ref, bv_ref,
                    wp_ref, bp_ref, pol_ref, val_ref):
    h = jnp.dot(x1_ref[...], w1a_ref[...], preferred_element_type=jnp.float32)
    h = h + jnp.dot(xd_ref[...], w1d_ref[...],
                    preferred_element_type=jnp.float32)
    h = jnp.maximum(h + b1_ref[...], 0.0)
    pol_ref[...] = jnp.dot(h, wp_ref[...],
                           preferred_element_type=jnp.float32) + bp_ref[...]
    val_ref[...] = jnp.tanh(
        jnp.dot(h, wv_ref[...], preferred_element_type=jnp.float32)
        + bv_ref[...])


def kernel(sparse_batch, dense_batch, stm_players, ft_w, ft_b, W1, b1, Wv, bv,
           Wp, bp):
    B, _ = sparse_batch.shape
    n_workers = 32
    per_w = B // n_workers
    sp_g = sparse_batch.reshape(n_workers, per_w, NSPARSE).transpose(0, 2, 1)
    stm_g = stm_players.reshape(n_workers, per_w)

    ft_out = _sc_ft_kernel(B, n_workers)(sp_g, stm_g, ft_w, ft_b)

    blk = 1024
    grid = (B // blk,)
    hid = W1.shape[1]
    ddim = dense_batch.shape[1]
    npol = Wp.shape[1]
    pol, val = pl.pallas_call(
        _tc_head_kernel,
        grid=grid,
        in_specs=[
            pl.BlockSpec((blk, 2 * FT_DIM), lambda i: (i, 0)),
            pl.BlockSpec((blk, ddim), lambda i: (i, 0)),
            pl.BlockSpec((2 * FT_DIM, hid), lambda i: (0, 0)),
            pl.BlockSpec((ddim, hid), lambda i: (0, 0)),
            pl.BlockSpec((1, hid), lambda i: (0, 0)),
            pl.BlockSpec((hid, 1), lambda i: (0, 0)),
            pl.BlockSpec((1, 1), lambda i: (0, 0)),
            pl.BlockSpec((hid, npol), lambda i: (0, 0)),
            pl.BlockSpec((1, npol), lambda i: (0, 0)),
        ],
        out_specs=[
            pl.BlockSpec((blk, npol), lambda i: (i, 0)),
            pl.BlockSpec((blk, 1), lambda i: (i, 0)),
        ],
        out_shape=[
            jax.ShapeDtypeStruct((B, npol), jnp.float32),
            jax.ShapeDtypeStruct((B, 1), jnp.float32),
        ],
    )(ft_out, dense_batch, W1[:2 * FT_DIM], W1[2 * FT_DIM:], b1[None], Wv,
      bv[None], Wp, bp[None])
    return pol, val[:, 0]
